# baseline (speedup 1.0000x reference)
import functools

import jax
import jax.numpy as jnp
import numpy as np
from jax import lax
from jax.experimental import pallas as pl
from jax.experimental.pallas import tpu as pltpu
from jax.experimental.pallas import tpu_sc as plsc

_B = 1024
_L = 200
_D = 128
_SCALE = float(np.sqrt(np.float32(_D)))

_NC = 2
_NS = 16
_NW = _NC * _NS
_SEQ_PER_W = _B // _NW
_NBUF = 3


def _positional_table():
    pos = np.arange(_L)[:, None].astype(np.float32)
    i = np.arange(_D)[None, :].astype(np.float32)
    angle_rates = 1.0 / np.power(
        10000.0, (2.0 * np.floor(i / 2.0)) / np.float32(_D))
    angles = pos * angle_rates
    pe = np.zeros((_L, _D), dtype=np.float32)
    pe[:, 0::2] = np.sin(angles[:, 0::2])
    pe[:, 1::2] = np.cos(angles[:, 1::2])
    return pe


_PE = _positional_table()


def _enc_kernel(idx_hbm, table_hbm, pe_hbm, out_hbm,
                pe_v, idx_t, w_idx, g0, g1, g2, gsem, wsem):
    wid = lax.axis_index("s") * _NC + lax.axis_index("c")
    rings = (g0, g1, g2)
    pltpu.sync_copy(pe_hbm, pe_v)
    pltpu.sync_copy(idx_hbm.at[wid], idx_t)

    iota = lax.iota(jnp.int32, 16)
    stride = iota * _L
    woff = wid * _SEQ_PER_W * _L

    def widx_body(l, c):
        for h in range(_SEQ_PER_W // 16):
            w_idx[l, pl.ds(h * 16, 16)] = stride + (woff + h * 16 * _L + l)
        return c

    lax.fori_loop(0, _L, widx_body, 0)

    def issue_gather(c, b):
        pltpu.async_copy(table_hbm.at[idx_t.at[c]], rings[b], gsem.at[b])

    def wait_gather(b):
        pltpu.make_async_copy(out_hbm.at[pl.ds(0, _SEQ_PER_W)], rings[b],
                              gsem.at[b]).wait()

    def issue_write(c, b):
        pltpu.async_copy(rings[b], out_hbm.at[w_idx.at[c]], wsem.at[b])

    def wait_write(b):
        pltpu.make_async_copy(rings[b], out_hbm.at[pl.ds(0, _SEQ_PER_W)],
                              wsem.at[b]).wait()

    def fuse(c, b):
        g = rings[b]
        pe_regs = [pe_v[c, pl.ds(d * 16, 16)] for d in range(_D // 16)]
        for j in range(_SEQ_PER_W):
            for d in range(_D // 16):
                sl = pl.ds(d * 16, 16)
                g[j, sl] = g[j, sl] * _SCALE + pe_regs[d]

    def step(c, b, first, last):
        if not first:
            wait_write((b + 1) % _NBUF)
        if not last:
            issue_gather(c + 1, (b + 1) % _NBUF)
        wait_gather(b)
        fuse(c, b)
        issue_write(c, b)

    issue_gather(0, 0)
    for u in range(_NBUF):
        step(u, u, u < _NBUF - 1, False)

    def group(gi, c):
        c0 = gi * _NBUF
        for u in range(_NBUF):
            step(c0 + u, u, False, False)
        return c

    lax.fori_loop(1, (_L - 2) // _NBUF, group, 0)
    step(_L - 2, (_L - 2) % _NBUF, False, False)
    step(_L - 1, (_L - 1) % _NBUF, False, True)
    wait_write((_L - 2) % _NBUF)
    wait_write((_L - 1) % _NBUF)


@jax.jit
def _encode(idx, emb_table, pe):
    mesh = plsc.VectorSubcoreMesh(core_axis_name="c", subcore_axis_name="s")
    f = functools.partial(
        pl.kernel,
        out_type=jax.ShapeDtypeStruct((_B * _L, _D), jnp.float32),
        mesh=mesh,
        scratch_types=[
            pltpu.VMEM((_L, _D), jnp.float32),
            pltpu.VMEM((_L, _SEQ_PER_W), jnp.int32),
            pltpu.VMEM((_L, _SEQ_PER_W), jnp.int32),
            pltpu.VMEM((_SEQ_PER_W, _D), jnp.float32),
            pltpu.VMEM((_SEQ_PER_W, _D), jnp.float32),
            pltpu.VMEM((_SEQ_PER_W, _D), jnp.float32),
            pltpu.SemaphoreType.DMA((_NBUF,)),
            pltpu.SemaphoreType.DMA((_NBUF,)),
        ],
    )(_enc_kernel)
    return f(idx, emb_table, pe)


def kernel(text, emb_table):
    idx = (text.astype(jnp.int32)
           .reshape(_NW, _SEQ_PER_W, _L)
           .transpose(0, 2, 1))
    out = _encode(idx, emb_table, _PE)
    return out.reshape(_B, _L, _D)


# speedup vs baseline: 1.2910x; 1.2910x over previous
import functools

import jax
import jax.numpy as jnp
import numpy as np
from jax import lax
from jax.experimental import pallas as pl
from jax.experimental.pallas import tpu as pltpu
from jax.experimental.pallas import tpu_sc as plsc

_B = 1024
_L = 200
_D = 128
_SCALE = float(np.sqrt(np.float32(_D)))

_NC = 2
_NS = 16
_NW = _NC * _NS
_SEQ_PER_W = _B // _NW
_K = 4
_NQ = 5
_QL = _L // _NQ
_NUNIT = (_SEQ_PER_W // _K) * _NQ
_NBUF = 3


def _positional_table():
    pos = np.arange(_L)[:, None].astype(np.float32)
    i = np.arange(_D)[None, :].astype(np.float32)
    angle_rates = 1.0 / np.power(
        10000.0, (2.0 * np.floor(i / 2.0)) / np.float32(_D))
    angles = pos * angle_rates
    pe = np.zeros((_L, _D), dtype=np.float32)
    pe[:, 0::2] = np.sin(angles[:, 0::2])
    pe[:, 1::2] = np.cos(angles[:, 1::2])
    return pe


_PE = _positional_table()


def _enc_kernel(idx_hbm, table_hbm, pe_hbm, out_hbm,
                pe_v, idx_v, *rest):
    bufs = rest[:_NBUF * _K]
    gsem, wsem = rest[_NBUF * _K:]
    wid = lax.axis_index("s") * _NC + lax.axis_index("c")
    pltpu.sync_copy(pe_hbm, pe_v)
    pltpu.sync_copy(idx_hbm.at[wid], idx_v)

    def unit_qr(c):
        return c // _NQ, lax.rem(c, _NQ)

    def issue_gathers(c, slot):
        q, r = unit_qr(c)
        for s in range(_K):
            pltpu.async_copy(table_hbm.at[idx_v.at[q * _K + s, r]],
                             bufs[slot * _K + s], gsem.at[slot])

    def wait_gathers(slot):
        for s in range(_K):
            pltpu.make_async_copy(out_hbm.at[pl.ds(0, _QL)],
                                  bufs[slot * _K + s], gsem.at[slot]).wait()

    def issue_writes(c, slot):
        q, r = unit_qr(c)
        for s in range(_K):
            base = (wid * _SEQ_PER_W + q * _K + s) * _L + r * _QL
            pltpu.async_copy(bufs[slot * _K + s],
                             out_hbm.at[pl.ds(base, _QL)], wsem.at[slot])

    def wait_writes(slot):
        for s in range(_K):
            pltpu.make_async_copy(bufs[slot * _K + s],
                                  out_hbm.at[pl.ds(0, _QL)],
                                  wsem.at[slot]).wait()

    def fuse(c, slot):
        _, r = unit_qr(c)
        roff = r * _QL

        def body(l, carry):
            for d in range(_D // 16):
                sl = pl.ds(d * 16, 16)
                pe_reg = pe_v[roff + l, sl]
                for s in range(_K):
                    g = bufs[slot * _K + s]
                    g[l, sl] = g[l, sl] * _SCALE + pe_reg
            return carry

        lax.fori_loop(0, _QL, body, 0)

    def step(c, slot, first, last):
        if not first:
            wait_writes((slot + 1) % _NBUF)
        if not last:
            issue_gathers(c + 1, (slot + 1) % _NBUF)
        wait_gathers(slot)
        fuse(c, slot)
        issue_writes(c, slot)

    issue_gathers(0, 0)
    for u in range(_NBUF):
        step(u, u, u < _NBUF - 1, False)

    def group(gi, carry):
        c0 = gi * _NBUF
        for u in range(_NBUF):
            step(c0 + u, u, False, False)
        return carry

    n_groups = _NUNIT // _NBUF - 2
    tail_start = (1 + n_groups) * _NBUF
    lax.fori_loop(1, 1 + n_groups, group, 0)
    for c in range(tail_start, _NUNIT):
        step(c, c % _NBUF, False, c == _NUNIT - 1)
    wait_writes((_NUNIT - 2) % _NBUF)
    wait_writes((_NUNIT - 1) % _NBUF)


@jax.jit
def _encode(idx, emb_table, pe):
    mesh = plsc.VectorSubcoreMesh(core_axis_name="c", subcore_axis_name="s")
    scratch = [
        pltpu.VMEM((_L, _D), jnp.float32),
        pltpu.VMEM((_SEQ_PER_W, _NQ, _QL), jnp.int32),
    ]
    scratch += [pltpu.VMEM((_QL, _D), jnp.float32)
                for _ in range(_NBUF * _K)]
    scratch += [
        pltpu.SemaphoreType.DMA((_NBUF,)),
        pltpu.SemaphoreType.DMA((_NBUF,)),
    ]
    f = functools.partial(
        pl.kernel,
        out_type=jax.ShapeDtypeStruct((_B * _L, _D), jnp.float32),
        mesh=mesh,
        scratch_types=scratch,
    )(_enc_kernel)
    return f(idx, emb_table, pe)


def kernel(text, emb_table):
    idx = text.astype(jnp.int32).reshape(_NW, _SEQ_PER_W, _NQ, _QL)
    out = _encode(idx, emb_table, _PE)
    return out.reshape(_B, _L, _D)


# speedup vs baseline: 1.4342x; 1.1109x over previous
import functools

import jax
import jax.numpy as jnp
import numpy as np
from jax import lax
from jax.experimental import pallas as pl
from jax.experimental.pallas import tpu as pltpu
from jax.experimental.pallas import tpu_sc as plsc

_B = 1024
_L = 200
_D = 128
_SCALE = float(np.sqrt(np.float32(_D)))

_NC = 2
_NS = 16
_NW = _NC * _NS
_SEQ_PER_W = _B // _NW
_NPAIR = _SEQ_PER_W // 2
_NBUF = 4


def _positional_table():
    pos = np.arange(_L)[:, None].astype(np.float32)
    i = np.arange(_D)[None, :].astype(np.float32)
    angle_rates = 1.0 / np.power(
        10000.0, (2.0 * np.floor(i / 2.0)) / np.float32(_D))
    angles = pos * angle_rates
    pe = np.zeros((_L, _D), dtype=np.float32)
    pe[:, 0::2] = np.sin(angles[:, 0::2])
    pe[:, 1::2] = np.cos(angles[:, 1::2])
    return pe


_PE = _positional_table()


def _enc_kernel(idx_hbm, table_hbm, pe_hbm, out_hbm,
                pe_v, i0, i1, i2, i3, r0, r1, r2, r3, gsem, wsem, isem):
    wid = lax.axis_index("s") * _NC + lax.axis_index("c")
    rows = (r0, r1, r2, r3)
    idxb = (i0, i1, i2, i3)
    pltpu.sync_copy(pe_hbm, pe_v)
    seq0 = wid * _SEQ_PER_W

    def issue_idx(j, b):
        pltpu.async_copy(idx_hbm.at[pl.ds((seq0 + j) * _L, _L)], idxb[b],
                         isem.at[b])

    def wait_idx(b):
        pltpu.make_async_copy(idx_hbm.at[pl.ds(0, _L)], idxb[b],
                              isem.at[b]).wait()

    def issue_gather(b):
        pltpu.async_copy(table_hbm.at[idxb[b].at[pl.ds(0, 128)]],
                         rows[b].at[pl.ds(0, 128)], gsem.at[b])
        pltpu.async_copy(table_hbm.at[idxb[b].at[pl.ds(128, _L - 128)]],
                         rows[b].at[pl.ds(128, _L - 128)], gsem.at[b])

    def wait_gather(b):
        pltpu.make_async_copy(out_hbm.at[pl.ds(0, _L)], rows[b],
                              gsem.at[b]).wait()

    def issue_write(j, b):
        pltpu.async_copy(rows[b], out_hbm.at[pl.ds((seq0 + j) * _L, _L)],
                         wsem.at[b])

    def wait_write(b):
        pltpu.make_async_copy(rows[b], out_hbm.at[pl.ds(0, _L)],
                              wsem.at[b]).wait()

    def fuse_pair(ba, bb):
        ga = rows[ba]
        gb = rows[bb]

        def body(l, carry):
            for d in range(_D // 16):
                sl = pl.ds(d * 16, 16)
                pe_reg = pe_v[l, sl]
                ga[l, sl] = ga[l, sl] * _SCALE + pe_reg
                gb[l, sl] = gb[l, sl] * _SCALE + pe_reg
            return carry

        lax.fori_loop(0, _L, body, 0)

    def pair_step(p, e0, e1, first, has_next, has_idx):
        o0, o1 = (e0 + 2) % _NBUF, (e1 + 2) % _NBUF
        if not first:
            wait_write(o0)
            wait_write(o1)
        if has_next:
            wait_idx(o0)
            wait_idx(o1)
            issue_gather(o0)
            issue_gather(o1)
        wait_gather(e0)
        wait_gather(e1)
        if has_idx:
            issue_idx(2 * p + 4, e0)
            issue_idx(2 * p + 5, e1)
        fuse_pair(e0, e1)
        issue_write(2 * p, e0)
        issue_write(2 * p + 1, e1)

    for b in range(_NBUF):
        issue_idx(b, b)
    wait_idx(0)
    wait_idx(1)
    issue_gather(0)
    issue_gather(1)

    pair_step(0, 0, 1, True, True, True)
    pair_step(1, 2, 3, False, True, True)

    def group(g2, carry):
        p0 = g2 * 2
        pair_step(p0, 0, 1, False, True, True)
        pair_step(p0 + 1, 2, 3, False, True, True)
        return carry

    lax.fori_loop(1, _NPAIR // 2 - 1, group, 0)
    pair_step(_NPAIR - 2, 0, 1, False, True, False)
    pair_step(_NPAIR - 1, 2, 3, False, False, False)
    wait_write(2)
    wait_write(3)


@jax.jit
def _encode(idx, emb_table, pe):
    mesh = plsc.VectorSubcoreMesh(core_axis_name="c", subcore_axis_name="s")
    f = functools.partial(
        pl.kernel,
        out_type=jax.ShapeDtypeStruct((_B * _L, _D), jnp.float32),
        mesh=mesh,
        scratch_types=[
            pltpu.VMEM((_L, _D), jnp.float32),
            pltpu.VMEM((_L,), jnp.int32),
            pltpu.VMEM((_L,), jnp.int32),
            pltpu.VMEM((_L,), jnp.int32),
            pltpu.VMEM((_L,), jnp.int32),
            pltpu.VMEM((_L, _D), jnp.float32),
            pltpu.VMEM((_L, _D), jnp.float32),
            pltpu.VMEM((_L, _D), jnp.float32),
            pltpu.VMEM((_L, _D), jnp.float32),
            pltpu.SemaphoreType.DMA((_NBUF,)),
            pltpu.SemaphoreType.DMA((_NBUF,)),
            pltpu.SemaphoreType.DMA((_NBUF,)),
        ],
    )(_enc_kernel)
    return f(idx, emb_table, pe)


def kernel(text, emb_table):
    idx = text.reshape(-1).astype(jnp.int32)
    out = _encode(idx, emb_table, _PE)
    return out.reshape(_B, _L, _D)


# speedup vs baseline: 1.4850x; 1.0354x over previous
import functools

import jax
import jax.numpy as jnp
import numpy as np
from jax import lax
from jax.experimental import pallas as pl
from jax.experimental.pallas import tpu as pltpu
from jax.experimental.pallas import tpu_sc as plsc

_B = 1024
_L = 200
_D = 128
_SCALE = float(np.sqrt(np.float32(_D)))

_NC = 2
_NS = 16
_NW = _NC * _NS
_SEQ_PER_W = _B // _NW
_NBUF = 3


def _positional_table():
    pos = np.arange(_L)[:, None].astype(np.float32)
    i = np.arange(_D)[None, :].astype(np.float32)
    angle_rates = 1.0 / np.power(
        10000.0, (2.0 * np.floor(i / 2.0)) / np.float32(_D))
    angles = pos * angle_rates
    pe = np.zeros((_L, _D), dtype=np.float32)
    pe[:, 0::2] = np.sin(angles[:, 0::2])
    pe[:, 1::2] = np.cos(angles[:, 1::2])
    return pe


_PE = _positional_table()


def _enc_kernel(idx_hbm, table_hbm, pe_hbm, out_hbm,
                pe_v, idx_v, r0, r1, r2, gsem, wsem):
    wid = lax.axis_index("s") * _NC + lax.axis_index("c")
    rows = (r0, r1, r2)
    pltpu.sync_copy(pe_hbm, pe_v)
    pltpu.sync_copy(idx_hbm.at[pl.ds(wid * _SEQ_PER_W * _L, _SEQ_PER_W * _L)],
                    idx_v)

    def issue_gather(j, b):
        off = j * _L
        pltpu.async_copy(table_hbm.at[idx_v.at[pl.ds(off, 128)]],
                         rows[b].at[pl.ds(0, 128)], gsem.at[b])
        pltpu.async_copy(table_hbm.at[idx_v.at[pl.ds(off + 128, _L - 128)]],
                         rows[b].at[pl.ds(128, _L - 128)], gsem.at[b])

    def wait_gather(b):
        pltpu.make_async_copy(out_hbm.at[pl.ds(0, _L)], rows[b],
                              gsem.at[b]).wait()

    def issue_write(j, b):
        pltpu.async_copy(rows[b], out_hbm.at[pl.ds((wid * _SEQ_PER_W + j) * _L,
                                                   _L)], wsem.at[b])

    def wait_write(b):
        pltpu.make_async_copy(rows[b], out_hbm.at[pl.ds(0, _L)],
                              wsem.at[b]).wait()

    def fuse(b):
        g = rows[b]

        def body(h, c):
            l = h * 2
            for u in range(2):
                for d in range(_D // 16):
                    sl = pl.ds(d * 16, 16)
                    g[l + u, sl] = g[l + u, sl] * _SCALE + pe_v[l + u, sl]
            return c
        lax.fori_loop(0, _L // 2, body, 0)

    def step(j, b, first, last):
        if not first:
            wait_write((b + 1) % _NBUF)
        if not last:
            issue_gather(j + 1, (b + 1) % _NBUF)
        wait_gather(b)
        fuse(b)
        issue_write(j, b)

    issue_gather(0, 0)

    def group(g, c):
        j0 = g * _NBUF
        for u in range(_NBUF):
            step(j0 + u, u, False, False)
        return c

    for u in range(_NBUF):
        step(u, u, u < _NBUF - 1, False)
    lax.fori_loop(1, _SEQ_PER_W // _NBUF, group, 0)
    step(_SEQ_PER_W - 2, (_SEQ_PER_W - 2) % _NBUF, False, False)
    step(_SEQ_PER_W - 1, (_SEQ_PER_W - 1) % _NBUF, False, True)
    wait_write((_SEQ_PER_W - 2) % _NBUF)
    wait_write((_SEQ_PER_W - 1) % _NBUF)


@jax.jit
def _encode(idx, emb_table, pe):
    mesh = plsc.VectorSubcoreMesh(core_axis_name="c", subcore_axis_name="s")
    f = functools.partial(
        pl.kernel,
        out_type=jax.ShapeDtypeStruct((_B * _L, _D), jnp.float32),
        mesh=mesh,
        scratch_types=[
            pltpu.VMEM((_L, _D), jnp.float32),
            pltpu.VMEM((_SEQ_PER_W * _L,), jnp.int32),
            pltpu.VMEM((_L, _D), jnp.float32),
            pltpu.VMEM((_L, _D), jnp.float32),
            pltpu.VMEM((_L, _D), jnp.float32),
            pltpu.SemaphoreType.DMA((_NBUF,)),
            pltpu.SemaphoreType.DMA((_NBUF,)),
        ],
    )(_enc_kernel)
    return f(idx, emb_table, pe)


def kernel(text, emb_table):
    idx = text.reshape(-1).astype(jnp.int32)
    out = _encode(idx, emb_table, _PE)
    return out.reshape(_B, _L, _D)
